# baseline (speedup 1.0000x reference)
import functools

import jax
import jax.numpy as jnp
from jax import lax
from jax.experimental import pallas as pl
from jax.experimental.pallas import tpu as pltpu
from jax.experimental.pallas import tpu_sc as plsc

N = 10000
E = 160000
D = 256
DH = D // 2

NC = 2
NS = 16
NW = NC * NS

EPW = E // NW
_HIST_PAD = -(-EPW // 16) * 16

NBLK = 10

@functools.partial(
    pl.kernel,
    out_type=jax.ShapeDtypeStruct((NBLK, NW, N // NBLK), jnp.float32),
    scratch_types=[
        pltpu.VMEM((_HIST_PAD,), jnp.int32),
        pltpu.VMEM((N,), jnp.float32),
    ],
    mesh=plsc.VectorSubcoreMesh(core_axis_name="c", subcore_axis_name="s"),
    compiler_params=pltpu.CompilerParams(
        needs_layout_passes=False, use_tc_tiling_on_sc=False),
)
def _sc_hist(dst_hbm, hist_out, idx_v, hist_v):
    c = lax.axis_index("c")
    s = lax.axis_index("s")
    wid = s * NC + c

    zeros16 = jnp.zeros((16,), jnp.float32)

    def zero_body(i, _):
        hist_v[pl.ds(i * 16, 16)] = zeros16
        return _

    lax.fori_loop(0, N // 16, zero_body, None)

    idx_v[pl.ds(_HIST_PAD - 16, 16)] = jnp.zeros((16,), jnp.int32)
    pltpu.sync_copy(dst_hbm.at[pl.ds(wid * EPW, EPW)], idx_v.at[pl.ds(0, EPW)])

    ones16 = jnp.ones((16,), jnp.float32)
    nfull = EPW // 16

    def hist_body(i, _):
        idx = idx_v[pl.ds(i * 16, 16)]
        plsc.addupdate_scatter(hist_v, [idx], ones16)
        return _

    lax.fori_loop(0, nfull, hist_body, None)

    rem = EPW - nfull * 16
    if rem:
        idx = idx_v[pl.ds(nfull * 16, 16)]
        mask = lax.iota(jnp.int32, 16) < rem
        plsc.addupdate_scatter(hist_v, [idx], ones16, mask=mask)

    rb = N // NBLK

    def wr_body(i, _):
        pltpu.sync_copy(hist_v.at[pl.ds(i * rb, rb)], hist_out.at[i, wid])
        return _

    lax.fori_loop(0, NBLK, wr_body, None)


EPT = E // NS
CH = 125
NCHUNK = EPT // CH
NQUAD = NCHUNK // 4
RPT = N // NS


@functools.partial(
    pl.kernel,
    out_type=jax.ShapeDtypeStruct((NC, N, DH), jnp.bfloat16),
    scratch_types=[
        pltpu.VMEM((NCHUNK, CH), jnp.int32),
        pltpu.VMEM((NCHUNK, CH), jnp.int32),
        pltpu.VMEM((CH, DH), jnp.bfloat16),
        pltpu.VMEM((CH, DH), jnp.bfloat16),
        pltpu.VMEM((CH, DH), jnp.bfloat16),
        pltpu.VMEM((CH, DH), jnp.bfloat16),
        pltpu.VMEM_SHARED((N, DH), jnp.bfloat16),
        pltpu.SemaphoreType.DMA,
        pltpu.SemaphoreType.DMA,
        pltpu.SemaphoreType.DMA,
    ],
    mesh=plsc.VectorSubcoreMesh(core_axis_name="c", subcore_axis_name="s"),
    compiler_params=pltpu.CompilerParams(
        needs_layout_passes=False, use_tc_tiling_on_sc=False),
)
def _sc_edges(gsrc_hbm, dst_hbm, y_hbm, acc_out,
              gidx, didx, rows0, rows1, rows2, rows3,
              acc, semi, semg, sems):
    c = lax.axis_index("c")
    s = lax.axis_index("s")
    bufs = (rows0, rows1, rows2, rows3)

    init_cp = pltpu.async_copy(y_hbm.at[pl.ds(c * N + s * RPT, RPT)],
                               acc.at[pl.ds(s * RPT, RPT)], semi)

    pltpu.sync_copy(gsrc_hbm.at[c * NS + s], gidx)
    pltpu.sync_copy(dst_hbm.at[s], didx)

    def start_gather(i, buf):
        pltpu.async_copy(y_hbm.at[gidx.at[i]], buf, semg)

    def wait_gather(i, buf):
        pltpu.make_async_copy(y_hbm.at[gidx.at[i]], buf, semg).wait()

    def start_scatter(i, buf):
        pltpu.async_copy(buf, acc.at[didx.at[i]], sems, add=True)

    def wait_scatter(i, buf):
        pltpu.make_async_copy(buf, acc.at[didx.at[i]], sems).wait()

    start_gather(0, rows0)
    start_gather(1, rows1)

    init_cp.wait()
    plsc.subcore_barrier()

    def quad_body(q, _):
        for j in range(4):
            i = 4 * q + j
            wait_gather(i, bufs[j])
            start_scatter(i, bufs[j])
            jp = (j + 2) % 4
            if j >= 2:
                wait_scatter(i - 2, bufs[jp])
            else:
                @pl.when(i >= 2)
                def _():
                    wait_scatter(i - 2, bufs[jp])

            @pl.when(i + 2 < NCHUNK)
            def _():
                start_gather(i + 2, bufs[jp])

        return _

    lax.fori_loop(0, NQUAD, quad_body, None)
    wait_scatter(NCHUNK - 2, bufs[2])
    wait_scatter(NCHUNK - 1, bufs[3])
    plsc.subcore_barrier()

    pltpu.sync_copy(acc.at[pl.ds(s * RPT, RPT)],
                    acc_out.at[c, pl.ds(s * RPT, RPT)])


RB = 1000


def _tc_linear_body(x_ref, w_ref, hist_ref, y_ref, dis_ref):
    xw = jnp.dot(x_ref[...], w_ref[...], preferred_element_type=jnp.float32)
    deg = 1.0 + jnp.sum(hist_ref[0], axis=0)
    dis = lax.rsqrt(deg)
    y = xw * dis[:, None]
    y_ref[0] = y[:, :DH]
    y_ref[1] = y[:, DH:]
    dis_ref[...] = dis[:, None]


def _tc_linear(x, W, hist):
    return pl.pallas_call(
        _tc_linear_body,
        grid=(N // RB,),
        in_specs=[
            pl.BlockSpec((RB, D), lambda i: (i, 0)),
            pl.BlockSpec((D, D), lambda i: (0, 0)),
            pl.BlockSpec((1, NW, RB), lambda i: (i, 0, 0)),
        ],
        out_specs=[
            pl.BlockSpec((NC, RB, DH), lambda i: (0, i, 0)),
            pl.BlockSpec((RB, 1), lambda i: (i, 0)),
        ],
        out_shape=[
            jax.ShapeDtypeStruct((NC, N, DH), jnp.float32),
            jax.ShapeDtypeStruct((N, 1), jnp.float32),
        ],
    )(x, W, hist)


def _tc_epilogue_body(acc_ref, dis_ref, b_ref, g_ref, be_ref, x_ref, o_ref):
    h = jnp.concatenate([acc_ref[0], acc_ref[1]], axis=-1)
    h = h * dis_ref[...] + b_ref[...]
    h = jnp.where(h > 0, h, jnp.exp(h) - 1.0)
    mean = jnp.mean(h, axis=-1, keepdims=True)
    var = jnp.mean((h - mean) ** 2, axis=-1, keepdims=True)
    h = (h - mean) * lax.rsqrt(var + 1e-5) * g_ref[...] + be_ref[...]
    o_ref[...] = h + x_ref[...]


def _tc_epilogue(acc, dis, b, gamma, beta, x):
    return pl.pallas_call(
        _tc_epilogue_body,
        grid=(N // RB,),
        in_specs=[
            pl.BlockSpec((NC, RB, DH), lambda i: (0, i, 0)),
            pl.BlockSpec((RB, 1), lambda i: (i, 0)),
            pl.BlockSpec((1, D), lambda i: (0, 0)),
            pl.BlockSpec((1, D), lambda i: (0, 0)),
            pl.BlockSpec((1, D), lambda i: (0, 0)),
            pl.BlockSpec((RB, D), lambda i: (i, 0)),
        ],
        out_specs=pl.BlockSpec((RB, D), lambda i: (i, 0)),
        out_shape=jax.ShapeDtypeStruct((N, D), jnp.float32),
    )(acc, dis, b, gamma, beta, x)


def kernel(x, edge_index, W, b, gamma, beta):
    src = edge_index[0]
    dst = edge_index[1]
    hist = _sc_hist(dst)
    y2, dis = _tc_linear(x, W, hist)
    yflat = y2.astype(jnp.bfloat16).reshape(NC * N, DH)
    gsrc = jnp.concatenate([src, src + N]).reshape(NC * NS, NCHUNK, CH)
    dst3 = dst.reshape(NS, NCHUNK, CH)
    acc = _sc_edges(gsrc, dst3, yflat)
    acc = acc.astype(jnp.float32)
    return _tc_epilogue(acc, dis,
                        b.reshape(1, D), gamma.reshape(1, D),
                        beta.reshape(1, D), x)


# speedup vs baseline: 1.0683x; 1.0683x over previous
import functools

import jax
import jax.numpy as jnp
from jax import lax
from jax.experimental import pallas as pl
from jax.experimental.pallas import tpu as pltpu
from jax.experimental.pallas import tpu_sc as plsc

N = 10000
E = 160000
D = 256
DH = D // 2

NC = 2
NS = 16
NW = NC * NS

EPW = E // NW
_HIST_PAD = -(-EPW // 16) * 16

NBLK = 5

@functools.partial(
    pl.kernel,
    out_type=jax.ShapeDtypeStruct((NBLK, NW, N // NBLK), jnp.float32),
    scratch_types=[
        pltpu.VMEM((_HIST_PAD,), jnp.int32),
        pltpu.VMEM((N,), jnp.float32),
    ],
    mesh=plsc.VectorSubcoreMesh(core_axis_name="c", subcore_axis_name="s"),
    compiler_params=pltpu.CompilerParams(
        needs_layout_passes=False, use_tc_tiling_on_sc=False),
)
def _sc_hist(dst_hbm, hist_out, idx_v, hist_v):
    c = lax.axis_index("c")
    s = lax.axis_index("s")
    wid = s * NC + c

    zeros16 = jnp.zeros((16,), jnp.float32)

    def zero_body(i, _):
        hist_v[pl.ds(i * 16, 16)] = zeros16
        return _

    lax.fori_loop(0, N // 16, zero_body, None)

    idx_v[pl.ds(_HIST_PAD - 16, 16)] = jnp.zeros((16,), jnp.int32)
    pltpu.sync_copy(dst_hbm.at[pl.ds(wid * EPW, EPW)], idx_v.at[pl.ds(0, EPW)])

    ones16 = jnp.ones((16,), jnp.float32)
    nfull = EPW // 16

    def hist_body(i, _):
        idx = idx_v[pl.ds(i * 16, 16)]
        plsc.addupdate_scatter(hist_v, [idx], ones16)
        return _

    lax.fori_loop(0, nfull, hist_body, None)

    rem = EPW - nfull * 16
    if rem:
        idx = idx_v[pl.ds(nfull * 16, 16)]
        mask = lax.iota(jnp.int32, 16) < rem
        plsc.addupdate_scatter(hist_v, [idx], ones16, mask=mask)

    rb = N // NBLK

    def wr_body(i, _):
        pltpu.sync_copy(hist_v.at[pl.ds(i * rb, rb)], hist_out.at[i, wid])
        return _

    lax.fori_loop(0, NBLK, wr_body, None)


EPT = E // NS
CH = 125
NCHUNK = EPT // CH
NQUAD = NCHUNK // 4
RPT = N // NS


@functools.partial(
    pl.kernel,
    out_type=jax.ShapeDtypeStruct((NC, N, DH), jnp.bfloat16),
    scratch_types=[
        pltpu.VMEM((NCHUNK, CH), jnp.int32),
        pltpu.VMEM((NCHUNK, CH), jnp.int32),
        pltpu.VMEM((CH, DH), jnp.bfloat16),
        pltpu.VMEM((CH, DH), jnp.bfloat16),
        pltpu.VMEM((CH, DH), jnp.bfloat16),
        pltpu.VMEM((CH, DH), jnp.bfloat16),
        pltpu.VMEM_SHARED((N, DH), jnp.bfloat16),
        pltpu.SemaphoreType.DMA,
        pltpu.SemaphoreType.DMA,
        pltpu.SemaphoreType.DMA,
    ],
    mesh=plsc.VectorSubcoreMesh(core_axis_name="c", subcore_axis_name="s"),
    compiler_params=pltpu.CompilerParams(
        needs_layout_passes=False, use_tc_tiling_on_sc=False),
)
def _sc_edges(gsrc_hbm, dst_hbm, y_hbm, acc_out,
              gidx, didx, rows0, rows1, rows2, rows3,
              acc, semi, semg, sems):
    c = lax.axis_index("c")
    s = lax.axis_index("s")
    bufs = (rows0, rows1, rows2, rows3)

    init_cp = pltpu.async_copy(y_hbm.at[pl.ds(c * N + s * RPT, RPT)],
                               acc.at[pl.ds(s * RPT, RPT)], semi)

    pltpu.sync_copy(gsrc_hbm.at[c * NS + s], gidx)
    pltpu.sync_copy(dst_hbm.at[s], didx)

    def start_gather(i, buf):
        pltpu.async_copy(y_hbm.at[gidx.at[i]], buf, semg)

    def wait_gather(i, buf):
        pltpu.make_async_copy(y_hbm.at[gidx.at[i]], buf, semg).wait()

    def start_scatter(i, buf):
        pltpu.async_copy(buf, acc.at[didx.at[i]], sems, add=True)

    def wait_scatter(i, buf):
        pltpu.make_async_copy(buf, acc.at[didx.at[i]], sems).wait()

    start_gather(0, rows0)
    start_gather(1, rows1)

    init_cp.wait()
    plsc.subcore_barrier()

    def quad_body(q, _):
        for j in range(4):
            i = 4 * q + j
            wait_gather(i, bufs[j])
            start_scatter(i, bufs[j])
            jp = (j + 2) % 4
            if j >= 2:
                wait_scatter(i - 2, bufs[jp])
            else:
                @pl.when(i >= 2)
                def _():
                    wait_scatter(i - 2, bufs[jp])

            @pl.when(i + 2 < NCHUNK)
            def _():
                start_gather(i + 2, bufs[jp])

        return _

    lax.fori_loop(0, NQUAD, quad_body, None)
    wait_scatter(NCHUNK - 2, bufs[2])
    wait_scatter(NCHUNK - 1, bufs[3])
    plsc.subcore_barrier()

    pltpu.sync_copy(acc.at[pl.ds(s * RPT, RPT)],
                    acc_out.at[c, pl.ds(s * RPT, RPT)])


RB = 2000


def _tc_linear_body(x_ref, w_ref, hist_ref, y_ref, dis_ref):
    xw = jnp.dot(x_ref[...], w_ref[...], preferred_element_type=jnp.float32)
    deg = 1.0 + jnp.sum(hist_ref[0], axis=0)
    dis = lax.rsqrt(deg)
    y = (xw * dis[:, None]).astype(jnp.bfloat16)
    y_ref[0] = y[:, :DH]
    y_ref[1] = y[:, DH:]
    dis_ref[...] = dis[:, None]


def _tc_linear(x, W, hist):
    return pl.pallas_call(
        _tc_linear_body,
        grid=(N // RB,),
        in_specs=[
            pl.BlockSpec((RB, D), lambda i: (i, 0)),
            pl.BlockSpec((D, D), lambda i: (0, 0)),
            pl.BlockSpec((1, NW, RB), lambda i: (i, 0, 0)),
        ],
        out_specs=[
            pl.BlockSpec((NC, RB, DH), lambda i: (0, i, 0)),
            pl.BlockSpec((RB, 1), lambda i: (i, 0)),
        ],
        out_shape=[
            jax.ShapeDtypeStruct((NC, N, DH), jnp.bfloat16),
            jax.ShapeDtypeStruct((N, 1), jnp.float32),
        ],
    )(x, W, hist)


def _tc_epilogue_body(acc_ref, dis_ref, b_ref, g_ref, be_ref, x_ref, o_ref):
    h = jnp.concatenate([acc_ref[0], acc_ref[1]], axis=-1).astype(jnp.float32)
    h = h * dis_ref[...] + b_ref[...]
    h = jnp.where(h > 0, h, jnp.exp(h) - 1.0)
    mean = jnp.mean(h, axis=-1, keepdims=True)
    var = jnp.mean((h - mean) ** 2, axis=-1, keepdims=True)
    h = (h - mean) * lax.rsqrt(var + 1e-5) * g_ref[...] + be_ref[...]
    o_ref[...] = h + x_ref[...]


def _tc_epilogue(acc, dis, b, gamma, beta, x):
    return pl.pallas_call(
        _tc_epilogue_body,
        grid=(N // RB,),
        in_specs=[
            pl.BlockSpec((NC, RB, DH), lambda i: (0, i, 0)),
            pl.BlockSpec((RB, 1), lambda i: (i, 0)),
            pl.BlockSpec((1, D), lambda i: (0, 0)),
            pl.BlockSpec((1, D), lambda i: (0, 0)),
            pl.BlockSpec((1, D), lambda i: (0, 0)),
            pl.BlockSpec((RB, D), lambda i: (i, 0)),
        ],
        out_specs=pl.BlockSpec((RB, D), lambda i: (i, 0)),
        out_shape=jax.ShapeDtypeStruct((N, D), jnp.float32),
    )(acc, dis, b, gamma, beta, x)


def kernel(x, edge_index, W, b, gamma, beta):
    src = edge_index[0]
    dst = edge_index[1]
    hist = _sc_hist(dst)
    y2, dis = _tc_linear(x, W, hist)
    yflat = y2.reshape(NC * N, DH)
    gsrc = jnp.concatenate([src, src + N]).reshape(NC * NS, NCHUNK, CH)
    dst3 = dst.reshape(NS, NCHUNK, CH)
    acc = _sc_edges(gsrc, dst3, yflat)
    return _tc_epilogue(acc, dis,
                        b.reshape(1, D), gamma.reshape(1, D),
                        beta.reshape(1, D), x)
